# Initial kernel scaffold; baseline (speedup 1.0000x reference)
import functools

import jax
import jax.numpy as jnp
from jax import lax
from jax.experimental import pallas as pl
from jax.experimental.pallas import tpu as pltpu
from jax.experimental.pallas import tpu_sc as plsc

N = 10000
E = 320000
D = 128

NC = 2
NS = 16
NW = NC * NS
K = 128

N_PAD = 10016
RPT = N_PAD // NS
NBLK = -(-E // (NW * K))
E_PAD = NW * NBLK * K
EPW = NBLK * K

_mesh = plsc.VectorSubcoreMesh(core_axis_name="c", subcore_axis_name="s")


@functools.partial(
    pl.kernel,
    out_type=jax.ShapeDtypeStruct((NC, 2, N_PAD, 8), jnp.float32),
    mesh=_mesh,
    scratch_types=[
        pltpu.VMEM((NBLK, K), jnp.int32),
        pltpu.VMEM((NBLK, K), jnp.int32),
        pltpu.VMEM((K, 8), jnp.float32),
        pltpu.VMEM_SHARED((N_PAD, 8), jnp.float32),
        pltpu.VMEM_SHARED((N_PAD, 8), jnp.float32),
    ],
)
def _deg_kernel(src_hbm, dst_hbm, ones_hbm, zeros_hbm, out_hbm,
                src_v, dst_v, ones_v, acc_out, acc_in):
    cid = lax.axis_index("c")
    sid = lax.axis_index("s")
    wid = cid * NS + sid

    rows = pl.ds(sid * RPT, RPT)
    pltpu.sync_copy(zeros_hbm, acc_out.at[rows])
    pltpu.sync_copy(zeros_hbm, acc_in.at[rows])
    pltpu.sync_copy(ones_hbm, ones_v)
    pltpu.sync_copy(src_hbm.at[wid], src_v)
    pltpu.sync_copy(dst_hbm.at[wid], dst_v)
    plsc.subcore_barrier()

    def body(j, carry):
        pltpu.sync_copy(ones_v, acc_out.at[src_v.at[j]], add=True)
        pltpu.sync_copy(ones_v, acc_in.at[dst_v.at[j]], add=True)
        return carry

    lax.fori_loop(0, NBLK, body, 0)
    plsc.subcore_barrier()

    pltpu.sync_copy(acc_out.at[rows], out_hbm.at[cid, 0, rows])
    pltpu.sync_copy(acc_in.at[rows], out_hbm.at[cid, 1, rows])


@functools.partial(
    pl.kernel,
    out_type=jax.ShapeDtypeStruct((NC, N_PAD, D), jnp.float32),
    mesh=_mesh,
    scratch_types=[
        pltpu.VMEM((NBLK, K), jnp.int32),
        pltpu.VMEM((NBLK, K), jnp.int32),
        pltpu.VMEM((K, D), jnp.float32),
        pltpu.VMEM_SHARED((N_PAD, D), jnp.float32),
        pltpu.SemaphoreType.DMA,
    ],
)
def _agg_kernel(y_hbm, src_hbm, dst_hbm, zeros_hbm, out_hbm,
                src_v, dst_v, rows_v, acc, sem):
    cid = lax.axis_index("c")
    sid = lax.axis_index("s")
    wid = cid * NS + sid

    rows = pl.ds(sid * RPT, RPT)
    pltpu.sync_copy(zeros_hbm, acc.at[rows])
    pltpu.sync_copy(src_hbm.at[wid], src_v)
    pltpu.sync_copy(dst_hbm.at[wid], dst_v)
    plsc.subcore_barrier()

    def body(j, carry):
        pltpu.async_copy(y_hbm.at[src_v.at[j]], rows_v, sem).wait()
        pltpu.sync_copy(rows_v, acc.at[dst_v.at[j]], add=True)
        return carry

    lax.fori_loop(0, NBLK, body, 0)
    plsc.subcore_barrier()

    pltpu.sync_copy(acc.at[rows], out_hbm.at[cid, rows])


def _scale_vecs(degp):
    dego = degp[0, 0, :, 0:1] + degp[1, 0, :, 0:1]
    degi = degp[0, 1, :, 0:1] + degp[1, 1, :, 0:1]
    a = lax.rsqrt(jnp.maximum(dego, 1.0))
    c = lax.rsqrt(jnp.maximum(degi, 1.0))
    return a, c


def _tc1_body(degp_ref, x_ref, w_ref, y_ref):
    a, _ = _scale_vecs(degp_ref[...])
    xw = jnp.dot(x_ref[...], w_ref[...], preferred_element_type=jnp.float32)
    y_ref[...] = xw * a


def _tc2_body(degp_ref, p_ref, b_ref, w_ref, y_ref):
    a, c = _scale_vecs(degp_ref[...])
    h = jnp.maximum((p_ref[0] + p_ref[1]) * c + b_ref[...], 0.0)
    y_ref[...] = jnp.dot(h, w_ref[...], preferred_element_type=jnp.float32) * a


def _tc3_body(degp_ref, p_ref, b_ref, o_ref):
    _, c = _scale_vecs(degp_ref[...])
    o_ref[...] = (p_ref[0] + p_ref[1]) * c + b_ref[...]


_f32 = jnp.float32
_tc1 = pl.pallas_call(_tc1_body, out_shape=jax.ShapeDtypeStruct((N_PAD, D), _f32))
_tc2 = pl.pallas_call(_tc2_body, out_shape=jax.ShapeDtypeStruct((N_PAD, D), _f32))
_tc3 = pl.pallas_call(_tc3_body, out_shape=jax.ShapeDtypeStruct((N_PAD, D), _f32))


@jax.jit
def kernel(G, x, W1, b1, W2, b2):
    src = G[0]
    dst = G[1]
    pad = jnp.full((E_PAD - E,), N, dtype=jnp.int32)
    src3 = jnp.concatenate([src, pad]).reshape(NW, NBLK, K)
    dst3 = jnp.concatenate([dst, pad]).reshape(NW, NBLK, K)
    x_pad = jnp.zeros((N_PAD, D), _f32).at[:N].set(x)

    ones8 = jnp.zeros((K, 8), _f32).at[:, 0].set(1.0)
    zeros8 = jnp.zeros((RPT, 8), _f32)
    zrows = jnp.zeros((RPT, D), _f32)

    degp = _deg_kernel(src3, dst3, ones8, zeros8)
    y1 = _tc1(degp, x_pad, W1)
    p1 = _agg_kernel(y1, src3, dst3, zrows)
    y2 = _tc2(degp, p1, b1.reshape(1, D), W2)
    p2 = _agg_kernel(y2, src3, dst3, zrows)
    out = _tc3(degp, p2, b2.reshape(1, D))
    return out[:N]


# speedup vs baseline: 12.0013x; 12.0013x over previous
import functools

import jax
import jax.numpy as jnp
from jax import lax
from jax.experimental import pallas as pl
from jax.experimental.pallas import tpu as pltpu
from jax.experimental.pallas import tpu_sc as plsc

N = 10000
E = 320000
D = 128

NC = 2
NS = 16
NW = NC * NS
K = 128

N_PAD = 10112
RPT = N_PAD // NS
NBLK = -(-E // (NW * K))
E_PAD = NW * NBLK * K
EPW = NBLK * K

_mesh = plsc.VectorSubcoreMesh(core_axis_name="c", subcore_axis_name="s")


@functools.partial(
    pl.kernel,
    out_type=jax.ShapeDtypeStruct((NC, 2, N_PAD, 8), jnp.float32),
    mesh=_mesh,
    scratch_types=[
        pltpu.VMEM((NBLK, K), jnp.int32),
        pltpu.VMEM((NBLK, K), jnp.int32),
        pltpu.VMEM((K, 8), jnp.float32),
        pltpu.VMEM_SHARED((N_PAD, 8), jnp.float32),
        pltpu.VMEM_SHARED((N_PAD, 8), jnp.float32),
    ],
)
def _deg_kernel(src_hbm, dst_hbm, ones_hbm, zeros_hbm, out_hbm,
                src_v, dst_v, ones_v, acc_out, acc_in):
    cid = lax.axis_index("c")
    sid = lax.axis_index("s")
    wid = cid * NS + sid

    rows = pl.ds(sid * RPT, RPT)
    pltpu.sync_copy(zeros_hbm, acc_out.at[rows])
    pltpu.sync_copy(zeros_hbm, acc_in.at[rows])
    pltpu.sync_copy(ones_hbm, ones_v)
    pltpu.sync_copy(src_hbm.at[wid], src_v)
    pltpu.sync_copy(dst_hbm.at[wid], dst_v)
    plsc.subcore_barrier()

    def body(j, carry):
        pltpu.sync_copy(ones_v, acc_out.at[src_v.at[j]], add=True)
        pltpu.sync_copy(ones_v, acc_in.at[dst_v.at[j]], add=True)
        return carry

    lax.fori_loop(0, NBLK, body, 0)
    plsc.subcore_barrier()

    pltpu.sync_copy(acc_out.at[rows], out_hbm.at[cid, 0, rows])
    pltpu.sync_copy(acc_in.at[rows], out_hbm.at[cid, 1, rows])


@functools.partial(
    pl.kernel,
    out_type=jax.ShapeDtypeStruct((NC, N_PAD, D), jnp.float32),
    mesh=_mesh,
    scratch_types=[
        pltpu.VMEM((NBLK, K), jnp.int32),
        pltpu.VMEM((NBLK, K), jnp.int32),
        pltpu.VMEM((K, D), jnp.float32),
        pltpu.VMEM_SHARED((N_PAD, D), jnp.float32),
        pltpu.SemaphoreType.DMA,
    ],
)
def _agg_kernel(y_hbm, src_hbm, dst_hbm, zeros_hbm, out_hbm,
                src_v, dst_v, rows_v, acc, sem):
    cid = lax.axis_index("c")
    sid = lax.axis_index("s")
    wid = cid * NS + sid

    rows = pl.ds(sid * RPT, RPT)
    pltpu.sync_copy(zeros_hbm, acc.at[rows])
    pltpu.sync_copy(src_hbm.at[wid], src_v)
    pltpu.sync_copy(dst_hbm.at[wid], dst_v)
    plsc.subcore_barrier()

    def body(j, carry):
        pltpu.async_copy(y_hbm.at[src_v.at[j]], rows_v, sem).wait()
        pltpu.sync_copy(rows_v, acc.at[dst_v.at[j]], add=True)
        return carry

    lax.fori_loop(0, NBLK, body, 0)
    plsc.subcore_barrier()

    pltpu.sync_copy(acc.at[rows], out_hbm.at[cid, rows])


def _scale_vecs(degp):
    dego = degp[0, 0, :, 0:1] + degp[1, 0, :, 0:1]
    degi = degp[0, 1, :, 0:1] + degp[1, 1, :, 0:1]
    a = lax.rsqrt(jnp.maximum(dego, 1.0))
    c = lax.rsqrt(jnp.maximum(degi, 1.0))
    return a, c


def _tc1_body(degp_ref, x_ref, w_ref, y_ref):
    a, _ = _scale_vecs(degp_ref[...])
    xw = jnp.dot(x_ref[...], w_ref[...], preferred_element_type=jnp.float32)
    y_ref[...] = xw * a


def _tc2_body(degp_ref, p_ref, b_ref, w_ref, y_ref):
    a, c = _scale_vecs(degp_ref[...])
    h = jnp.maximum((p_ref[0] + p_ref[1]) * c + b_ref[...], 0.0)
    y_ref[...] = jnp.dot(h, w_ref[...], preferred_element_type=jnp.float32) * a


def _tc3_body(degp_ref, p_ref, b_ref, o_ref):
    _, c = _scale_vecs(degp_ref[...])
    o_ref[...] = (p_ref[0] + p_ref[1]) * c + b_ref[...]


_f32 = jnp.float32
_tc1 = pl.pallas_call(_tc1_body, out_shape=jax.ShapeDtypeStruct((N_PAD, D), _f32))
_tc2 = pl.pallas_call(_tc2_body, out_shape=jax.ShapeDtypeStruct((N_PAD, D), _f32))
_tc3 = pl.pallas_call(_tc3_body, out_shape=jax.ShapeDtypeStruct((N_PAD, D), _f32))


@jax.jit
def kernel(G, x, W1, b1, W2, b2):
    src = G[0]
    dst = G[1]
    pad = jnp.full((E_PAD - E,), N, dtype=jnp.int32)
    src3 = jnp.concatenate([src, pad]).reshape(NW, NBLK, K)
    dst3 = jnp.concatenate([dst, pad]).reshape(NW, NBLK, K)
    x_pad = jnp.zeros((N_PAD, D), _f32).at[:N].set(x)

    ones8 = jnp.zeros((K, 8), _f32).at[:, 0].set(1.0)
    zeros8 = jnp.zeros((RPT, 8), _f32)
    zrows = jnp.zeros((RPT, D), _f32)

    degp = _deg_kernel(src3, dst3, ones8, zeros8)
    y1 = _tc1(degp, x_pad, W1)
    p1 = _agg_kernel(y1, src3, dst3, zrows)
    y2 = _tc2(degp, p1, b1.reshape(1, D), W2)
    p2 = _agg_kernel(y2, src3, dst3, zrows)
    out = _tc3(degp, p2, b2.reshape(1, D))
    return out[:N]
